# baseline (speedup 1.0000x reference)
import functools

import jax
import jax.numpy as jnp
from jax import lax
from jax.experimental import pallas as pl
from jax.experimental.pallas import tpu as pltpu
from jax.experimental.pallas import tpu_sc as plsc

S, H, NH, DH = 2048, 768, 12, 64
E, F, TOPK = 8, 3072, 2
_DEBUG_JNP_COMBINE = False
_DEBUG_JNP_XS = False
TILE = 256
NTILES = 24
NP = NTILES * TILE


def _qkv_body(x_ref, w_ref, b_ref, wq_ref, bq_ref, wk_ref, bk_ref,
              wv_ref, bv_ref, q_ref, k_ref, v_ref):
    x = x_ref[...]
    m = jnp.mean(x, axis=-1, keepdims=True)
    var = jnp.mean((x - m) ** 2, axis=-1, keepdims=True)
    h = (x - m) / jnp.sqrt(var + 1e-5) * w_ref[...] + b_ref[...]
    q_ref[...] = jnp.dot(h, wq_ref[...], preferred_element_type=jnp.float32) + bq_ref[...]
    k_ref[...] = jnp.dot(h, wk_ref[...], preferred_element_type=jnp.float32) + bk_ref[...]
    v_ref[...] = jnp.dot(h, wv_ref[...], preferred_element_type=jnp.float32) + bv_ref[...]


_RB = 512


def _qkv(x2d, ln1_w, ln1_b, Wq, bq, Wk, bk, Wv, bv):
    row = pl.BlockSpec((_RB, H), lambda j: (j, 0))
    vec = pl.BlockSpec((H,), lambda j: (0,))
    mat = pl.BlockSpec((H, H), lambda j: (0, 0))
    return pl.pallas_call(
        _qkv_body,
        grid=(S // _RB,),
        in_specs=[row, vec, vec, mat, vec, mat, vec, mat, vec],
        out_specs=[row, row, row],
        out_shape=[jax.ShapeDtypeStruct((S, H), jnp.float32)] * 3,
    )(x2d, ln1_w, ln1_b, Wq, bq, Wk, bk, Wv, bv)


_QB = 512


def _attn_body(q_ref, k_ref, v_ref, o_ref):
    for h in range(2):
        q = q_ref[:, h * DH:(h + 1) * DH]
        k = k_ref[:, h * DH:(h + 1) * DH]
        s = jax.lax.dot_general(q, k, (((1,), (1,)), ((), ())),
                                preferred_element_type=jnp.float32)
        s = s * (1.0 / 8.0)
        mx = jnp.max(s, axis=-1, keepdims=True)
        e = jnp.exp(s - mx)
        denom = jnp.sum(e, axis=-1, keepdims=True)
        o = jnp.dot(e, v_ref[:, h * DH:(h + 1) * DH],
                    preferred_element_type=jnp.float32)
        o_ref[:, h * DH:(h + 1) * DH] = o / denom


def _attn(q, k, v):
    kv_spec = pl.BlockSpec((S, 2 * DH), lambda j, qb: (0, j))
    q_spec = pl.BlockSpec((_QB, 2 * DH), lambda j, qb: (qb, j))
    return pl.pallas_call(
        _attn_body,
        grid=(NH // 2, S // _QB),
        in_specs=[q_spec, kv_spec, kv_spec],
        out_specs=q_spec,
        out_shape=jax.ShapeDtypeStruct((S, H), jnp.float32),
    )(q, k, v)


def _post_body(x_ref, o_ref, wo_ref, bo_ref, w_ref, b_ref, wr_ref, br_ref,
               x1_ref, h2_ref, lg_ref):
    x1 = x_ref[...] + jnp.dot(o_ref[...], wo_ref[...],
                              preferred_element_type=jnp.float32) + bo_ref[...]
    x1_ref[...] = x1
    m = jnp.mean(x1, axis=-1, keepdims=True)
    var = jnp.mean((x1 - m) ** 2, axis=-1, keepdims=True)
    h2 = (x1 - m) / jnp.sqrt(var + 1e-5) * w_ref[...] + b_ref[...]
    h2_ref[...] = h2
    lg_ref[...] = jnp.dot(h2, wr_ref[...], preferred_element_type=jnp.float32) + br_ref[...]


def _post(x2d, o, Wo, bo, ln2_w, ln2_b, Wr, br):
    row = pl.BlockSpec((_RB, H), lambda j: (j, 0))
    vec = pl.BlockSpec((H,), lambda j: (0,))
    return pl.pallas_call(
        _post_body,
        grid=(S // _RB,),
        in_specs=[row, row, pl.BlockSpec((H, H), lambda j: (0, 0)), vec,
                  vec, vec, pl.BlockSpec((H, E), lambda j: (0, 0)),
                  pl.BlockSpec((E,), lambda j: (0,))],
        out_specs=[row, row, pl.BlockSpec((_RB, E), lambda j: (j, 0))],
        out_shape=[
            jax.ShapeDtypeStruct((S, H), jnp.float32),
            jax.ShapeDtypeStruct((S, H), jnp.float32),
            jax.ShapeDtypeStruct((S, E), jnp.float32),
        ],
    )(x2d, o, Wo, bo, ln2_w, ln2_b, Wr, br)


TPW = S // 16


def _route_sc(logitsT, h2):
    mesh = plsc.VectorSubcoreMesh(core_axis_name="c", subcore_axis_name="s")

    @functools.partial(
        pl.kernel, mesh=mesh,
        out_type=[
            jax.ShapeDtypeStruct((NP, H), jnp.float32),
            jax.ShapeDtypeStruct((TOPK, S), jnp.int32),
            jax.ShapeDtypeStruct((TOPK, S), jnp.float32),
            jax.ShapeDtypeStruct((32,), jnp.int32),
            jax.ShapeDtypeStruct((32,), jnp.int32),
        ],
        compiler_params=pltpu.CompilerParams(
            needs_layout_passes=False, use_tc_tiling_on_sc=False),
        scratch_types=[
            pltpu.VMEM((E, TPW), jnp.float32),
            pltpu.VMEM((TOPK, TPW), jnp.int32),
            pltpu.VMEM((TOPK, TPW), jnp.int32),
            pltpu.VMEM((TOPK, TPW), jnp.float32),
            pltpu.VMEM((16,), jnp.int32),
            pltpu.VMEM((16, 16), jnp.int32),
            pltpu.VMEM((32, H), jnp.float32),
            pltpu.VMEM((2, 32), jnp.int32),
            pltpu.VMEM((2, 16), jnp.int32),
            pltpu.VMEM((2, 16), jnp.int32),
            pltpu.VMEM_SHARED((16, 16), jnp.int32),
            pltpu.SemaphoreType.DMA,
        ])
    def route(logT_hbm, h2_hbm, xs_hbm, pos_hbm, g_hbm, te_hbm, us_hbm,
              lg_v, e_v, p_v, g_v, cnt_v, call_v, h2_v, idx_v, te_v, us_v,
              cnt_sh, sem):
        c = lax.axis_index("c")
        s = lax.axis_index("s")

        @pl.when(c == 0)
        def _body():
            t0 = s * TPW
            iota = lax.iota(jnp.int32, 16)
            for e in range(E):
                pltpu.sync_copy(logT_hbm.at[e, pl.ds(t0, TPW)], lg_v.at[e])
            cnt = [jnp.int32(0)] * E
            for g in range(TPW // 16):
                sl = pl.ds(g * 16, 16)
                vs = [lg_v[e, sl] for e in range(E)]
                m1 = vs[0]
                a1 = jnp.zeros((16,), jnp.int32)
                for e in range(1, E):
                    gt = vs[e] > m1
                    m1 = jnp.where(gt, vs[e], m1)
                    a1 = jnp.where(gt, e, a1)
                m2 = jnp.full((16,), -3e38, jnp.float32)
                a2 = jnp.zeros((16,), jnp.int32)
                for e in range(E):
                    ve = jnp.where(a1 == e, -3e38, vs[e])
                    gt = ve > m2
                    m2 = jnp.where(gt, ve, m2)
                    a2 = jnp.where(gt, e, a2)
                ed = jnp.exp(m2 - m1)
                den = 1.0 + ed
                e_v[0, sl] = a1
                e_v[1, sl] = a2
                g_v[0, sl] = 1.0 / den
                g_v[1, sl] = ed / den
                for e in range(E):
                    cnt[e] = cnt[e] + jnp.sum((a1 == e).astype(jnp.int32)) \
                                    + jnp.sum((a2 == e).astype(jnp.int32))
            cv = jnp.zeros((16,), jnp.int32)
            for e in range(E):
                cv = cv + jnp.where(iota == e, cnt[e], 0)
            cnt_v[...] = cv
            pltpu.sync_copy(cnt_v, cnt_sh.at[s])
            plsc.subcore_barrier()
            pltpu.sync_copy(cnt_sh, call_v)
            s_vec = jnp.zeros((16,), jnp.int32) + s
            total = jnp.zeros((16,), jnp.int32)
            before = jnp.zeros((16,), jnp.int32)
            for w2 in range(16):
                r = call_v[w2]
                total = total + r
                mask = (jnp.zeros((16,), jnp.int32) + w2) < s_vec
                before = before + jnp.where(mask, r, 0)
            padded = jnp.bitwise_and(total + (TILE - 1), ~(TILE - 1))
            incl = plsc.cumsum(padded)
            pstart = incl - padded
            base = pstart + before

            @pl.when(s == 0)
            def _tiles():
                tot = jnp.sum(jnp.where(iota == E - 1, incl, 0))
                for grp in range(2):
                    jv = (iota + grp * 16) * TILE
                    te = jnp.zeros((16,), jnp.int32)
                    for e in range(E):
                        ps_e = jnp.sum(jnp.where(iota == e, pstart, 0))
                        ie_e = jnp.sum(jnp.where(iota == e, incl, 0))
                        te = jnp.where((jv >= ps_e) & (jv < ie_e), e, te)
                    te_v[grp, :] = jnp.where(jv >= tot, E - 1, te)
                    us_v[grp, :] = (jv < tot).astype(jnp.int32)
                pltpu.sync_copy(te_v.at[0], te_hbm.at[pl.ds(0, 16)])
                pltpu.sync_copy(te_v.at[1], te_hbm.at[pl.ds(16, 16)])
                pltpu.sync_copy(us_v.at[0], us_hbm.at[pl.ds(0, 16)])
                pltpu.sync_copy(us_v.at[1], us_hbm.at[pl.ds(16, 16)])

            off = [jnp.int32(0)] * E
            for g in range(TPW // 16):
                sl = pl.ds(g * 16, 16)
                for j2 in range(TOPK):
                    ev = e_v[j2, sl]
                    p = jnp.zeros((16,), jnp.int32)
                    for e in range(E):
                        mask = ev == e
                        mi = mask.astype(jnp.int32)
                        ranks = plsc.cumsum(mi) - 1
                        base_e = jnp.sum(jnp.where(iota == e, base, 0))
                        p = jnp.where(mask, base_e + off[e] + ranks, p)
                        off[e] = off[e] + jnp.sum(mi)
                    p_v[j2, sl] = p
            pltpu.sync_copy(p_v.at[0], pos_hbm.at[0, pl.ds(t0, TPW)])
            pltpu.sync_copy(p_v.at[1], pos_hbm.at[1, pl.ds(t0, TPW)])
            pltpu.sync_copy(g_v.at[0], g_hbm.at[0, pl.ds(t0, TPW)])
            pltpu.sync_copy(g_v.at[1], g_hbm.at[1, pl.ds(t0, TPW)])
            for ch in range(TPW // 32):
                pltpu.sync_copy(h2_hbm.at[pl.ds(t0 + ch * 32, 32)], h2_v)
                for j2 in range(TOPK):
                    for hh in range(2):
                        idx_v[j2, pl.ds(hh * 16, 16)] = \
                            p_v[j2, pl.ds(ch * 32 + hh * 16, 16)]
                    pltpu.async_copy(h2_v, xs_hbm.at[idx_v.at[j2]], sem).wait()

    return route(logitsT, h2)


CPW = S // 32


def _combine_sc(outp, x1, gates, pos):
    mesh = plsc.VectorSubcoreMesh(core_axis_name="c", subcore_axis_name="s")

    @functools.partial(
        pl.kernel, mesh=mesh,
        out_type=jax.ShapeDtypeStruct((S, H), jnp.float32),
        compiler_params=pltpu.CompilerParams(needs_layout_passes=False),
        scratch_types=[
            pltpu.VMEM((TOPK, CPW), jnp.int32),
            pltpu.VMEM((TOPK, CPW), jnp.float32),
            pltpu.VMEM((16,), jnp.int32),
            pltpu.VMEM((16,), jnp.int32),
            pltpu.VMEM((16, H), jnp.float32),
            pltpu.VMEM((16, H), jnp.float32),
            pltpu.VMEM((16, H), jnp.float32),
            pltpu.VMEM((16, H), jnp.float32),
            pltpu.SemaphoreType.DMA,
        ])
    def combine(outp_hbm, x1_hbm, g_hbm, pos_hbm, out_hbm,
                pos_v, g_v, i1, i2, c1, c2, x1v, ov, sem):
        c = lax.axis_index("c")
        s = lax.axis_index("s")
        wid = s * 2 + c
        t0 = wid * CPW
        pltpu.sync_copy(pos_hbm.at[0, pl.ds(t0, CPW)], pos_v.at[0])
        pltpu.sync_copy(pos_hbm.at[1, pl.ds(t0, CPW)], pos_v.at[1])
        pltpu.sync_copy(g_hbm.at[0, pl.ds(t0, CPW)], g_v.at[0])
        pltpu.sync_copy(g_hbm.at[1, pl.ds(t0, CPW)], g_v.at[1])
        for q in range(CPW // 16):
            i1[...] = pos_v[0, pl.ds(q * 16, 16)]
            i2[...] = pos_v[1, pl.ds(q * 16, 16)]
            pltpu.async_copy(outp_hbm.at[i1], c1, sem).wait()
            pltpu.async_copy(outp_hbm.at[i2], c2, sem).wait()
            pltpu.sync_copy(x1_hbm.at[pl.ds(t0 + q * 16, 16)], x1v)
            gq1 = g_v[0, pl.ds(q * 16, 16)]
            gq2 = g_v[1, pl.ds(q * 16, 16)]
            for i in range(16):
                g1 = gq1[i]
                g2 = gq2[i]

                def body(kk, _):
                    rsl = pl.ds(kk * 16, 16)
                    ov[i, rsl] = x1v[i, rsl] + g1 * c1[i, rsl] + g2 * c2[i, rsl]
                    return 0

                lax.fori_loop(0, H // 16, body, 0)
            pltpu.sync_copy(ov, out_hbm.at[pl.ds(t0 + q * 16, 16)])

    return combine(outp, x1, gates, pos)


def _ffn_body(te_ref, used_ref, xs_ref, w1_ref, b1_ref, w2_ref, b2_ref, out_ref):
    j = pl.program_id(0)

    @pl.when(used_ref[j] > 0)
    def _():
        xs = xs_ref[...].astype(jnp.bfloat16)
        w1 = w1_ref[0].astype(jnp.bfloat16)
        hid = jnp.dot(xs, w1, preferred_element_type=jnp.float32) + b1_ref[0]
        act = jax.nn.gelu(hid).astype(jnp.bfloat16)
        w2 = w2_ref[0].astype(jnp.bfloat16)
        out_ref[...] = jnp.dot(act, w2, preferred_element_type=jnp.float32) + b2_ref[0]


def _ffn(xs, W1, b1, W2, b2, tile_expert, tile_used):
    grid_spec = pltpu.PrefetchScalarGridSpec(
        num_scalar_prefetch=2,
        grid=(NTILES,),
        in_specs=[
            pl.BlockSpec((TILE, H), lambda j, te, us: (j, 0)),
            pl.BlockSpec((1, H, F), lambda j, te, us: (te[j], 0, 0)),
            pl.BlockSpec((1, 1, F), lambda j, te, us: (te[j], 0, 0)),
            pl.BlockSpec((1, F, H), lambda j, te, us: (te[j], 0, 0)),
            pl.BlockSpec((1, 1, H), lambda j, te, us: (te[j], 0, 0)),
        ],
        out_specs=pl.BlockSpec((TILE, H), lambda j, te, us: (j, 0)),
    )
    return pl.pallas_call(
        _ffn_body,
        grid_spec=grid_spec,
        out_shape=jax.ShapeDtypeStruct((NP, H), jnp.float32),
    )(tile_expert, tile_used, xs, W1, b1.reshape(E, 1, F), W2, b2.reshape(E, 1, H))


def kernel(x, ln1_w, ln1_b, Wq, bq, Wk, bk, Wv, bv, Wo, bo,
           ln2_w, ln2_b, Wr, br, W1, b1, W2, b2):
    x2d = x.reshape(S, H)
    q, k, v = _qkv(x2d, ln1_w, ln1_b, Wq, bq, Wk, bk, Wv, bv)
    o = _attn(q, k, v)
    x1, h2, logits = _post(x2d, o, Wo, bo, ln2_w, ln2_b, Wr, br)
    xs, pos, gates, tile_expert, tile_used = _route_sc(logits.T, h2)
    if _DEBUG_JNP_XS:
        xs = (jnp.zeros((NP, H), jnp.float32)
              .at[pos[0]].set(h2).at[pos[1]].set(h2))
    outp = _ffn(xs, W1, b1, W2, b2, tile_expert[:NTILES], tile_used[:NTILES])
    if _DEBUG_JNP_COMBINE:
        c1 = jnp.take(outp, pos[0], axis=0)
        c2 = jnp.take(outp, pos[1], axis=0)
        out = x1 + gates[0][:, None] * c1 + gates[1][:, None] * c2
    else:
        out = _combine_sc(outp, x1, gates, pos)
    return out.reshape(1, S, H)


# speedup vs baseline: 1.0064x; 1.0064x over previous
import functools

import jax
import jax.numpy as jnp
from jax import lax
from jax.experimental import pallas as pl
from jax.experimental.pallas import tpu as pltpu
from jax.experimental.pallas import tpu_sc as plsc

S, H, NH, DH = 2048, 768, 12, 64
E, F, TOPK = 8, 3072, 2
_DEBUG_JNP_COMBINE = False
_DEBUG_JNP_XS = False
TILE = 256
NTILES = 24
NP = NTILES * TILE


def _qkv_body(x_ref, w_ref, b_ref, wq_ref, bq_ref, wk_ref, bk_ref,
              wv_ref, bv_ref, q_ref, k_ref, v_ref):
    x = x_ref[...]
    m = jnp.mean(x, axis=-1, keepdims=True)
    var = jnp.mean((x - m) ** 2, axis=-1, keepdims=True)
    h = (x - m) / jnp.sqrt(var + 1e-5) * w_ref[...] + b_ref[...]
    q_ref[...] = jnp.dot(h, wq_ref[...], preferred_element_type=jnp.float32) + bq_ref[...]
    k_ref[...] = jnp.dot(h, wk_ref[...], preferred_element_type=jnp.float32) + bk_ref[...]
    v_ref[...] = jnp.dot(h, wv_ref[...], preferred_element_type=jnp.float32) + bv_ref[...]


_RB = 512


def _qkv(x2d, ln1_w, ln1_b, Wq, bq, Wk, bk, Wv, bv):
    row = pl.BlockSpec((_RB, H), lambda j: (j, 0))
    vec = pl.BlockSpec((H,), lambda j: (0,))
    mat = pl.BlockSpec((H, H), lambda j: (0, 0))
    return pl.pallas_call(
        _qkv_body,
        grid=(S // _RB,),
        in_specs=[row, vec, vec, mat, vec, mat, vec, mat, vec],
        out_specs=[row, row, row],
        out_shape=[jax.ShapeDtypeStruct((S, H), jnp.float32)] * 3,
    )(x2d, ln1_w, ln1_b, Wq, bq, Wk, bk, Wv, bv)


_QB = 512


def _attn_body(q_ref, k_ref, v_ref, o_ref):
    for h in range(2):
        q = q_ref[:, h * DH:(h + 1) * DH]
        k = k_ref[:, h * DH:(h + 1) * DH]
        s = jax.lax.dot_general(q, k, (((1,), (1,)), ((), ())),
                                preferred_element_type=jnp.float32)
        s = s * (1.0 / 8.0)
        mx = jnp.max(s, axis=-1, keepdims=True)
        e = jnp.exp(s - mx)
        denom = jnp.sum(e, axis=-1, keepdims=True)
        o = jnp.dot(e, v_ref[:, h * DH:(h + 1) * DH],
                    preferred_element_type=jnp.float32)
        o_ref[:, h * DH:(h + 1) * DH] = o / denom


def _attn(q, k, v):
    kv_spec = pl.BlockSpec((S, 2 * DH), lambda j, qb: (0, j))
    q_spec = pl.BlockSpec((_QB, 2 * DH), lambda j, qb: (qb, j))
    return pl.pallas_call(
        _attn_body,
        grid=(NH // 2, S // _QB),
        in_specs=[q_spec, kv_spec, kv_spec],
        out_specs=q_spec,
        out_shape=jax.ShapeDtypeStruct((S, H), jnp.float32),
    )(q, k, v)


def _post_body(x_ref, o_ref, wo_ref, bo_ref, w_ref, b_ref, wr_ref, br_ref,
               x1_ref, h2_ref, lg_ref):
    x1 = x_ref[...] + jnp.dot(o_ref[...], wo_ref[...],
                              preferred_element_type=jnp.float32) + bo_ref[...]
    x1_ref[...] = x1
    m = jnp.mean(x1, axis=-1, keepdims=True)
    var = jnp.mean((x1 - m) ** 2, axis=-1, keepdims=True)
    h2 = (x1 - m) / jnp.sqrt(var + 1e-5) * w_ref[...] + b_ref[...]
    h2_ref[...] = h2
    lg_ref[...] = jnp.dot(h2, wr_ref[...], preferred_element_type=jnp.float32) + br_ref[...]


def _post(x2d, o, Wo, bo, ln2_w, ln2_b, Wr, br):
    row = pl.BlockSpec((_RB, H), lambda j: (j, 0))
    vec = pl.BlockSpec((H,), lambda j: (0,))
    return pl.pallas_call(
        _post_body,
        grid=(S // _RB,),
        in_specs=[row, row, pl.BlockSpec((H, H), lambda j: (0, 0)), vec,
                  vec, vec, pl.BlockSpec((H, E), lambda j: (0, 0)),
                  pl.BlockSpec((E,), lambda j: (0,))],
        out_specs=[row, row, pl.BlockSpec((_RB, E), lambda j: (j, 0))],
        out_shape=[
            jax.ShapeDtypeStruct((S, H), jnp.float32),
            jax.ShapeDtypeStruct((S, H), jnp.float32),
            jax.ShapeDtypeStruct((S, E), jnp.float32),
        ],
    )(x2d, o, Wo, bo, ln2_w, ln2_b, Wr, br)


TPW = S // 16


def _route_sc(logitsT, h2):
    mesh = plsc.VectorSubcoreMesh(core_axis_name="c", subcore_axis_name="s")

    @functools.partial(
        pl.kernel, mesh=mesh,
        out_type=[
            jax.ShapeDtypeStruct((NP, H), jnp.float32),
            jax.ShapeDtypeStruct((TOPK, S), jnp.int32),
            jax.ShapeDtypeStruct((TOPK, S), jnp.float32),
            jax.ShapeDtypeStruct((32,), jnp.int32),
            jax.ShapeDtypeStruct((32,), jnp.int32),
        ],
        compiler_params=pltpu.CompilerParams(
            needs_layout_passes=False, use_tc_tiling_on_sc=False),
        scratch_types=[
            pltpu.VMEM((E, TPW), jnp.float32),
            pltpu.VMEM((TOPK, TPW), jnp.int32),
            pltpu.VMEM((TOPK, TPW), jnp.int32),
            pltpu.VMEM((TOPK, TPW), jnp.float32),
            pltpu.VMEM((16,), jnp.int32),
            pltpu.VMEM((16, 16), jnp.int32),
            pltpu.VMEM((32, H), jnp.float32),
            pltpu.VMEM((2, 32), jnp.int32),
            pltpu.VMEM((2, 16), jnp.int32),
            pltpu.VMEM((2, 16), jnp.int32),
            pltpu.VMEM_SHARED((16, 16), jnp.int32),
            pltpu.SemaphoreType.DMA,
        ])
    def route(logT_hbm, h2_hbm, xs_hbm, pos_hbm, g_hbm, te_hbm, us_hbm,
              lg_v, e_v, p_v, g_v, cnt_v, call_v, h2_v, idx_v, te_v, us_v,
              cnt_sh, sem):
        c = lax.axis_index("c")
        s = lax.axis_index("s")

        @pl.when(c == 0)
        def _body():
            t0 = s * TPW
            iota = lax.iota(jnp.int32, 16)
            for e in range(E):
                pltpu.sync_copy(logT_hbm.at[e, pl.ds(t0, TPW)], lg_v.at[e])
            cnt = [jnp.int32(0)] * E
            for g in range(TPW // 16):
                sl = pl.ds(g * 16, 16)
                vs = [lg_v[e, sl] for e in range(E)]
                m1 = vs[0]
                a1 = jnp.zeros((16,), jnp.int32)
                for e in range(1, E):
                    gt = vs[e] > m1
                    m1 = jnp.where(gt, vs[e], m1)
                    a1 = jnp.where(gt, e, a1)
                m2 = jnp.full((16,), -3e38, jnp.float32)
                a2 = jnp.zeros((16,), jnp.int32)
                for e in range(E):
                    ve = jnp.where(a1 == e, -3e38, vs[e])
                    gt = ve > m2
                    m2 = jnp.where(gt, ve, m2)
                    a2 = jnp.where(gt, e, a2)
                ed = jnp.exp(m2 - m1)
                den = 1.0 + ed
                e_v[0, sl] = a1
                e_v[1, sl] = a2
                g_v[0, sl] = 1.0 / den
                g_v[1, sl] = ed / den
                for e in range(E):
                    cnt[e] = cnt[e] + jnp.sum((a1 == e).astype(jnp.int32)) \
                                    + jnp.sum((a2 == e).astype(jnp.int32))
            cv = jnp.zeros((16,), jnp.int32)
            for e in range(E):
                cv = cv + jnp.where(iota == e, cnt[e], 0)
            cnt_v[...] = cv
            pltpu.sync_copy(cnt_v, cnt_sh.at[s])
            plsc.subcore_barrier()
            pltpu.sync_copy(cnt_sh, call_v)
            s_vec = jnp.zeros((16,), jnp.int32) + s
            total = jnp.zeros((16,), jnp.int32)
            before = jnp.zeros((16,), jnp.int32)
            for w2 in range(16):
                r = call_v[w2]
                total = total + r
                mask = (jnp.zeros((16,), jnp.int32) + w2) < s_vec
                before = before + jnp.where(mask, r, 0)
            padded = jnp.bitwise_and(total + (TILE - 1), ~(TILE - 1))
            incl = plsc.cumsum(padded)
            pstart = incl - padded
            base = pstart + before

            @pl.when(s == 0)
            def _tiles():
                tot = jnp.sum(jnp.where(iota == E - 1, incl, 0))
                for grp in range(2):
                    jv = (iota + grp * 16) * TILE
                    te = jnp.zeros((16,), jnp.int32)
                    for e in range(E):
                        ps_e = jnp.sum(jnp.where(iota == e, pstart, 0))
                        ie_e = jnp.sum(jnp.where(iota == e, incl, 0))
                        te = jnp.where((jv >= ps_e) & (jv < ie_e), e, te)
                    te_v[grp, :] = jnp.where(jv >= tot, E - 1, te)
                    us_v[grp, :] = (jv < tot).astype(jnp.int32)
                pltpu.sync_copy(te_v.at[0], te_hbm.at[pl.ds(0, 16)])
                pltpu.sync_copy(te_v.at[1], te_hbm.at[pl.ds(16, 16)])
                pltpu.sync_copy(us_v.at[0], us_hbm.at[pl.ds(0, 16)])
                pltpu.sync_copy(us_v.at[1], us_hbm.at[pl.ds(16, 16)])

            off = [jnp.int32(0)] * E
            for g in range(TPW // 16):
                sl = pl.ds(g * 16, 16)
                for j2 in range(TOPK):
                    ev = e_v[j2, sl]
                    p = jnp.zeros((16,), jnp.int32)
                    for e in range(E):
                        mask = ev == e
                        mi = mask.astype(jnp.int32)
                        ranks = plsc.cumsum(mi) - 1
                        base_e = jnp.sum(jnp.where(iota == e, base, 0))
                        p = jnp.where(mask, base_e + off[e] + ranks, p)
                        off[e] = off[e] + jnp.sum(mi)
                    p_v[j2, sl] = p
            pltpu.sync_copy(p_v.at[0], pos_hbm.at[0, pl.ds(t0, TPW)])
            pltpu.sync_copy(p_v.at[1], pos_hbm.at[1, pl.ds(t0, TPW)])
            pltpu.sync_copy(g_v.at[0], g_hbm.at[0, pl.ds(t0, TPW)])
            pltpu.sync_copy(g_v.at[1], g_hbm.at[1, pl.ds(t0, TPW)])
            for ch in range(TPW // 32):
                pltpu.sync_copy(h2_hbm.at[pl.ds(t0 + ch * 32, 32)], h2_v)
                for j2 in range(TOPK):
                    for hh in range(2):
                        idx_v[j2, pl.ds(hh * 16, 16)] = \
                            p_v[j2, pl.ds(ch * 32 + hh * 16, 16)]
                    pltpu.async_copy(h2_v, xs_hbm.at[idx_v.at[j2]], sem).wait()

    return route(logitsT, h2)


CPW = S // 32


def _combine_sc(outp, x1, gates, pos):
    mesh = plsc.VectorSubcoreMesh(core_axis_name="c", subcore_axis_name="s")

    @functools.partial(
        pl.kernel, mesh=mesh,
        out_type=jax.ShapeDtypeStruct((S, H), jnp.float32),
        compiler_params=pltpu.CompilerParams(needs_layout_passes=False),
        scratch_types=[
            pltpu.VMEM((TOPK, CPW), jnp.int32),
            pltpu.VMEM((TOPK, CPW), jnp.float32),
            pltpu.VMEM((16,), jnp.int32),
            pltpu.VMEM((16,), jnp.int32),
            pltpu.VMEM((16, H), jnp.float32),
            pltpu.VMEM((16, H), jnp.float32),
            pltpu.VMEM((16, H), jnp.float32),
            pltpu.VMEM((16, H), jnp.float32),
            pltpu.SemaphoreType.DMA,
        ])
    def combine(outp_hbm, x1_hbm, g_hbm, pos_hbm, out_hbm,
                pos_v, g_v, i1, i2, c1, c2, x1v, ov, sem):
        c = lax.axis_index("c")
        s = lax.axis_index("s")
        wid = s * 2 + c
        t0 = wid * CPW
        pltpu.sync_copy(pos_hbm.at[0, pl.ds(t0, CPW)], pos_v.at[0])
        pltpu.sync_copy(pos_hbm.at[1, pl.ds(t0, CPW)], pos_v.at[1])
        pltpu.sync_copy(g_hbm.at[0, pl.ds(t0, CPW)], g_v.at[0])
        pltpu.sync_copy(g_hbm.at[1, pl.ds(t0, CPW)], g_v.at[1])
        for q in range(CPW // 16):
            i1[...] = pos_v[0, pl.ds(q * 16, 16)]
            i2[...] = pos_v[1, pl.ds(q * 16, 16)]
            pltpu.async_copy(outp_hbm.at[i1], c1, sem).wait()
            pltpu.async_copy(outp_hbm.at[i2], c2, sem).wait()
            pltpu.sync_copy(x1_hbm.at[pl.ds(t0 + q * 16, 16)], x1v)
            gq1 = g_v[0, pl.ds(q * 16, 16)]
            gq2 = g_v[1, pl.ds(q * 16, 16)]
            for i in range(16):
                g1 = gq1[i]
                g2 = gq2[i]

                def body(kk, _):
                    rsl = pl.ds(kk * 16, 16)
                    ov[i, rsl] = x1v[i, rsl] + g1 * c1[i, rsl] + g2 * c2[i, rsl]
                    return 0

                lax.fori_loop(0, H // 16, body, 0)
            pltpu.sync_copy(ov, out_hbm.at[pl.ds(t0 + q * 16, 16)])

    return combine(outp, x1, gates, pos)


def _ffn_body(te_ref, used_ref, xs_ref, w1_ref, b1_ref, w2_ref, b2_ref, out_ref):
    j = pl.program_id(0)

    @pl.when(used_ref[j] > 0)
    def _():
        xs = xs_ref[...]
        hid = jnp.dot(xs, w1_ref[0], preferred_element_type=jnp.float32) + b1_ref[0]
        act = jax.nn.gelu(hid)
        out_ref[...] = jnp.dot(act, w2_ref[0], preferred_element_type=jnp.float32) + b2_ref[0]


def _ffn(xs, W1, b1, W2, b2, tile_expert, tile_used):
    grid_spec = pltpu.PrefetchScalarGridSpec(
        num_scalar_prefetch=2,
        grid=(NTILES,),
        in_specs=[
            pl.BlockSpec((TILE, H), lambda j, te, us: (j, 0)),
            pl.BlockSpec((1, H, F), lambda j, te, us: (te[j], 0, 0)),
            pl.BlockSpec((1, 1, F), lambda j, te, us: (te[j], 0, 0)),
            pl.BlockSpec((1, F, H), lambda j, te, us: (te[j], 0, 0)),
            pl.BlockSpec((1, 1, H), lambda j, te, us: (te[j], 0, 0)),
        ],
        out_specs=pl.BlockSpec((TILE, H), lambda j, te, us: (j, 0)),
    )
    return pl.pallas_call(
        _ffn_body,
        grid_spec=grid_spec,
        out_shape=jax.ShapeDtypeStruct((NP, H), jnp.float32),
    )(tile_expert, tile_used, xs, W1, b1.reshape(E, 1, F), W2, b2.reshape(E, 1, H))


def kernel(x, ln1_w, ln1_b, Wq, bq, Wk, bk, Wv, bv, Wo, bo,
           ln2_w, ln2_b, Wr, br, W1, b1, W2, b2):
    x2d = x.reshape(S, H)
    q, k, v = _qkv(x2d, ln1_w, ln1_b, Wq, bq, Wk, bk, Wv, bv)
    o = _attn(q, k, v)
    x1, h2, logits = _post(x2d, o, Wo, bo, ln2_w, ln2_b, Wr, br)
    xs, pos, gates, tile_expert, tile_used = _route_sc(logits.T, h2)
    if _DEBUG_JNP_XS:
        xs = (jnp.zeros((NP, H), jnp.float32)
              .at[pos[0]].set(h2).at[pos[1]].set(h2))
    outp = _ffn(xs, W1, b1, W2, b2, tile_expert[:NTILES], tile_used[:NTILES])
    if _DEBUG_JNP_COMBINE:
        c1 = jnp.take(outp, pos[0], axis=0)
        c2 = jnp.take(outp, pos[1], axis=0)
        out = x1 + gates[0][:, None] * c1 + gates[1][:, None] * c2
    else:
        out = _combine_sc(outp, x1, gates, pos)
    return out.reshape(1, S, H)


# speedup vs baseline: 1.0353x; 1.0287x over previous
import functools

import jax
import jax.numpy as jnp
from jax import lax
from jax.experimental import pallas as pl
from jax.experimental.pallas import tpu as pltpu
from jax.experimental.pallas import tpu_sc as plsc

S, H, NH, DH = 2048, 768, 12, 64
E, F, TOPK = 8, 3072, 2
_DEBUG_JNP_COMBINE = False
_DEBUG_JNP_XS = False
TILE = 256
NTILES = 24
NP = NTILES * TILE


def _qkv_body(x_ref, w_ref, b_ref, wq_ref, bq_ref, wk_ref, bk_ref,
              wv_ref, bv_ref, q_ref, k_ref, v_ref):
    x = x_ref[...]
    m = jnp.mean(x, axis=-1, keepdims=True)
    var = jnp.mean((x - m) ** 2, axis=-1, keepdims=True)
    h = (x - m) / jnp.sqrt(var + 1e-5) * w_ref[...] + b_ref[...]
    q_ref[...] = jnp.dot(h, wq_ref[...], preferred_element_type=jnp.float32) + bq_ref[...]
    k_ref[...] = jnp.dot(h, wk_ref[...], preferred_element_type=jnp.float32) + bk_ref[...]
    v_ref[...] = jnp.dot(h, wv_ref[...], preferred_element_type=jnp.float32) + bv_ref[...]


_RB = 512


def _qkv(x2d, ln1_w, ln1_b, Wq, bq, Wk, bk, Wv, bv):
    row = pl.BlockSpec((_RB, H), lambda j: (j, 0))
    vec = pl.BlockSpec((H,), lambda j: (0,))
    mat = pl.BlockSpec((H, H), lambda j: (0, 0))
    return pl.pallas_call(
        _qkv_body,
        grid=(S // _RB,),
        in_specs=[row, vec, vec, mat, vec, mat, vec, mat, vec],
        out_specs=[row, row, row],
        out_shape=[jax.ShapeDtypeStruct((S, H), jnp.float32)] * 3,
    )(x2d, ln1_w, ln1_b, Wq, bq, Wk, bk, Wv, bv)


_QB = 512


def _attn_body(q_ref, k_ref, v_ref, o_ref):
    for h in range(2):
        q = q_ref[:, h * DH:(h + 1) * DH]
        k = k_ref[:, h * DH:(h + 1) * DH]
        s = jax.lax.dot_general(q, k, (((1,), (1,)), ((), ())),
                                preferred_element_type=jnp.float32)
        s = s * (1.0 / 8.0)
        mx = jnp.max(s, axis=-1, keepdims=True)
        e = jnp.exp(s - mx)
        denom = jnp.sum(e, axis=-1, keepdims=True)
        o = jnp.dot(e, v_ref[:, h * DH:(h + 1) * DH],
                    preferred_element_type=jnp.float32)
        o_ref[:, h * DH:(h + 1) * DH] = o / denom


def _attn(q, k, v):
    kv_spec = pl.BlockSpec((S, 2 * DH), lambda j, qb: (0, j))
    q_spec = pl.BlockSpec((_QB, 2 * DH), lambda j, qb: (qb, j))
    return pl.pallas_call(
        _attn_body,
        grid=(NH // 2, S // _QB),
        in_specs=[q_spec, kv_spec, kv_spec],
        out_specs=q_spec,
        out_shape=jax.ShapeDtypeStruct((S, H), jnp.float32),
    )(q, k, v)


def _post_body(x_ref, o_ref, wo_ref, bo_ref, w_ref, b_ref, wr_ref, br_ref,
               x1_ref, h2_ref, lg_ref):
    x1 = x_ref[...] + jnp.dot(o_ref[...], wo_ref[...],
                              preferred_element_type=jnp.float32) + bo_ref[...]
    x1_ref[...] = x1
    m = jnp.mean(x1, axis=-1, keepdims=True)
    var = jnp.mean((x1 - m) ** 2, axis=-1, keepdims=True)
    h2 = (x1 - m) / jnp.sqrt(var + 1e-5) * w_ref[...] + b_ref[...]
    h2_ref[...] = h2
    lg_ref[...] = jnp.dot(h2, wr_ref[...], preferred_element_type=jnp.float32) + br_ref[...]


def _post(x2d, o, Wo, bo, ln2_w, ln2_b, Wr, br):
    row = pl.BlockSpec((_RB, H), lambda j: (j, 0))
    vec = pl.BlockSpec((H,), lambda j: (0,))
    return pl.pallas_call(
        _post_body,
        grid=(S // _RB,),
        in_specs=[row, row, pl.BlockSpec((H, H), lambda j: (0, 0)), vec,
                  vec, vec, pl.BlockSpec((H, E), lambda j: (0, 0)),
                  pl.BlockSpec((E,), lambda j: (0,))],
        out_specs=[row, row, pl.BlockSpec((_RB, E), lambda j: (j, 0))],
        out_shape=[
            jax.ShapeDtypeStruct((S, H), jnp.float32),
            jax.ShapeDtypeStruct((S, H), jnp.float32),
            jax.ShapeDtypeStruct((S, E), jnp.float32),
        ],
    )(x2d, o, Wo, bo, ln2_w, ln2_b, Wr, br)


TPW = S // 16


def _route_sc(logitsT, h2):
    mesh = plsc.VectorSubcoreMesh(core_axis_name="c", subcore_axis_name="s")

    @functools.partial(
        pl.kernel, mesh=mesh,
        out_type=[
            jax.ShapeDtypeStruct((NP, H), jnp.float32),
            jax.ShapeDtypeStruct((TOPK, S), jnp.int32),
            jax.ShapeDtypeStruct((TOPK, S), jnp.float32),
            jax.ShapeDtypeStruct((32,), jnp.int32),
            jax.ShapeDtypeStruct((32,), jnp.int32),
        ],
        compiler_params=pltpu.CompilerParams(
            needs_layout_passes=False, use_tc_tiling_on_sc=False),
        scratch_types=[
            pltpu.VMEM((E, TPW), jnp.float32),
            pltpu.VMEM((TOPK, TPW), jnp.int32),
            pltpu.VMEM((TOPK, TPW), jnp.int32),
            pltpu.VMEM((TOPK, TPW), jnp.float32),
            pltpu.VMEM((16,), jnp.int32),
            pltpu.VMEM((16, 16), jnp.int32),
            pltpu.VMEM((2, 32, H), jnp.float32),
            pltpu.VMEM((8, 32), jnp.int32),
            pltpu.VMEM((2, 16), jnp.int32),
            pltpu.VMEM((2, 16), jnp.int32),
            pltpu.VMEM_SHARED((16, 16), jnp.int32),
            pltpu.SemaphoreType.DMA,
        ])
    def route(logT_hbm, h2_hbm, xs_hbm, pos_hbm, g_hbm, te_hbm, us_hbm,
              lg_v, e_v, p_v, g_v, cnt_v, call_v, h2_v, idx_v, te_v, us_v,
              cnt_sh, sem):
        c = lax.axis_index("c")
        s = lax.axis_index("s")

        @pl.when(c == 0)
        def _body():
            t0 = s * TPW
            iota = lax.iota(jnp.int32, 16)
            for e in range(E):
                pltpu.sync_copy(logT_hbm.at[e, pl.ds(t0, TPW)], lg_v.at[e])
            cnt = [jnp.int32(0)] * E
            for g in range(TPW // 16):
                sl = pl.ds(g * 16, 16)
                vs = [lg_v[e, sl] for e in range(E)]
                m1 = vs[0]
                a1 = jnp.zeros((16,), jnp.int32)
                for e in range(1, E):
                    gt = vs[e] > m1
                    m1 = jnp.where(gt, vs[e], m1)
                    a1 = jnp.where(gt, e, a1)
                m2 = jnp.full((16,), -3e38, jnp.float32)
                a2 = jnp.zeros((16,), jnp.int32)
                for e in range(E):
                    ve = jnp.where(a1 == e, -3e38, vs[e])
                    gt = ve > m2
                    m2 = jnp.where(gt, ve, m2)
                    a2 = jnp.where(gt, e, a2)
                ed = jnp.exp(m2 - m1)
                den = 1.0 + ed
                e_v[0, sl] = a1
                e_v[1, sl] = a2
                g_v[0, sl] = 1.0 / den
                g_v[1, sl] = ed / den
                for e in range(E):
                    cnt[e] = cnt[e] + jnp.sum((a1 == e).astype(jnp.int32)) \
                                    + jnp.sum((a2 == e).astype(jnp.int32))
            cv = jnp.zeros((16,), jnp.int32)
            for e in range(E):
                cv = cv + jnp.where(iota == e, cnt[e], 0)
            cnt_v[...] = cv
            pltpu.sync_copy(cnt_v, cnt_sh.at[s])
            plsc.subcore_barrier()
            pltpu.sync_copy(cnt_sh, call_v)
            s_vec = jnp.zeros((16,), jnp.int32) + s
            total = jnp.zeros((16,), jnp.int32)
            before = jnp.zeros((16,), jnp.int32)
            for w2 in range(16):
                r = call_v[w2]
                total = total + r
                mask = (jnp.zeros((16,), jnp.int32) + w2) < s_vec
                before = before + jnp.where(mask, r, 0)
            padded = jnp.bitwise_and(total + (TILE - 1), ~(TILE - 1))
            incl = plsc.cumsum(padded)
            pstart = incl - padded
            base = pstart + before

            @pl.when(s == 0)
            def _tiles():
                tot = jnp.sum(jnp.where(iota == E - 1, incl, 0))
                for grp in range(2):
                    jv = (iota + grp * 16) * TILE
                    te = jnp.zeros((16,), jnp.int32)
                    for e in range(E):
                        ps_e = jnp.sum(jnp.where(iota == e, pstart, 0))
                        ie_e = jnp.sum(jnp.where(iota == e, incl, 0))
                        te = jnp.where((jv >= ps_e) & (jv < ie_e), e, te)
                    te_v[grp, :] = jnp.where(jv >= tot, E - 1, te)
                    us_v[grp, :] = (jv < tot).astype(jnp.int32)
                pltpu.sync_copy(te_v.at[0], te_hbm.at[pl.ds(0, 16)])
                pltpu.sync_copy(te_v.at[1], te_hbm.at[pl.ds(16, 16)])
                pltpu.sync_copy(us_v.at[0], us_hbm.at[pl.ds(0, 16)])
                pltpu.sync_copy(us_v.at[1], us_hbm.at[pl.ds(16, 16)])

            off = [jnp.int32(0)] * E
            for g in range(TPW // 16):
                sl = pl.ds(g * 16, 16)
                for j2 in range(TOPK):
                    ev = e_v[j2, sl]
                    p = jnp.zeros((16,), jnp.int32)
                    for e in range(E):
                        mask = ev == e
                        mi = mask.astype(jnp.int32)
                        ranks = plsc.cumsum(mi) - 1
                        base_e = jnp.sum(jnp.where(iota == e, base, 0))
                        p = jnp.where(mask, base_e + off[e] + ranks, p)
                        off[e] = off[e] + jnp.sum(mi)
                    p_v[j2, sl] = p
            pltpu.sync_copy(p_v.at[0], pos_hbm.at[0, pl.ds(t0, TPW)])
            pltpu.sync_copy(p_v.at[1], pos_hbm.at[1, pl.ds(t0, TPW)])
            pltpu.sync_copy(g_v.at[0], g_hbm.at[0, pl.ds(t0, TPW)])
            pltpu.sync_copy(g_v.at[1], g_hbm.at[1, pl.ds(t0, TPW)])
            handles = []
            for ch in range(TPW // 32):
                buf = h2_v.at[ch % 2]
                if ch >= 2:
                    handles.pop(0).wait()
                    handles.pop(0).wait()
                pltpu.sync_copy(h2_hbm.at[pl.ds(t0 + ch * 32, 32)], buf)
                for j2 in range(TOPK):
                    row = ch * 2 + j2
                    for hh in range(2):
                        idx_v[row, pl.ds(hh * 16, 16)] = \
                            p_v[j2, pl.ds(ch * 32 + hh * 16, 16)]
                    handles.append(
                        pltpu.async_copy(buf, xs_hbm.at[idx_v.at[row]], sem))
            for hc in handles:
                hc.wait()

    return route(logitsT, h2)


CPW = S // 32


def _combine_sc(outp, x1, gates, pos):
    mesh = plsc.VectorSubcoreMesh(core_axis_name="c", subcore_axis_name="s")

    @functools.partial(
        pl.kernel, mesh=mesh,
        out_type=jax.ShapeDtypeStruct((S, H), jnp.float32),
        compiler_params=pltpu.CompilerParams(needs_layout_passes=False),
        scratch_types=[
            pltpu.VMEM((TOPK, CPW), jnp.int32),
            pltpu.VMEM((TOPK, CPW), jnp.float32),
            pltpu.VMEM((2, 16), jnp.int32),
            pltpu.VMEM((2, 16), jnp.int32),
            pltpu.VMEM((2, 16, H), jnp.float32),
            pltpu.VMEM((2, 16, H), jnp.float32),
            pltpu.VMEM((16, H), jnp.float32),
            pltpu.VMEM((16, H), jnp.float32),
            pltpu.SemaphoreType.DMA,
        ])
    def combine(outp_hbm, x1_hbm, g_hbm, pos_hbm, out_hbm,
                pos_v, g_v, i1, i2, c1, c2, x1v, ov, sem):
        c = lax.axis_index("c")
        s = lax.axis_index("s")
        wid = s * 2 + c
        t0 = wid * CPW
        pltpu.sync_copy(pos_hbm.at[0, pl.ds(t0, CPW)], pos_v.at[0])
        pltpu.sync_copy(pos_hbm.at[1, pl.ds(t0, CPW)], pos_v.at[1])
        pltpu.sync_copy(g_hbm.at[0, pl.ds(t0, CPW)], g_v.at[0])
        pltpu.sync_copy(g_hbm.at[1, pl.ds(t0, CPW)], g_v.at[1])

        def fire(q):
            b = q % 2
            i1[b, :] = pos_v[0, pl.ds(q * 16, 16)]
            i2[b, :] = pos_v[1, pl.ds(q * 16, 16)]
            return [pltpu.async_copy(outp_hbm.at[i1.at[b]], c1.at[b], sem),
                    pltpu.async_copy(outp_hbm.at[i2.at[b]], c2.at[b], sem)]

        pend = fire(0)
        for q in range(CPW // 16):
            b = q % 2
            cur, pend = pend, (fire(q + 1) if q + 1 < CPW // 16 else [])
            for hc in cur:
                hc.wait()
            pltpu.sync_copy(x1_hbm.at[pl.ds(t0 + q * 16, 16)], x1v)
            gq1 = g_v[0, pl.ds(q * 16, 16)]
            gq2 = g_v[1, pl.ds(q * 16, 16)]
            for i in range(16):
                g1 = gq1[i]
                g2 = gq2[i]

                def body(kk, _):
                    rsl = pl.ds(kk * 16, 16)
                    ov[i, rsl] = (x1v[i, rsl] + g1 * c1[b, i, rsl]
                                  + g2 * c2[b, i, rsl])
                    return 0

                lax.fori_loop(0, H // 16, body, 0)
            pltpu.sync_copy(ov, out_hbm.at[pl.ds(t0 + q * 16, 16)])

    return combine(outp, x1, gates, pos)


def _ffn_body(te_ref, used_ref, xs_ref, w1_ref, b1_ref, w2_ref, b2_ref, out_ref):
    j = pl.program_id(0)

    @pl.when(used_ref[j] > 0)
    def _():
        xs = xs_ref[...]
        hid = jnp.dot(xs, w1_ref[0], preferred_element_type=jnp.float32) + b1_ref[0]
        act = jax.nn.gelu(hid)
        out_ref[...] = jnp.dot(act, w2_ref[0], preferred_element_type=jnp.float32) + b2_ref[0]


def _ffn(xs, W1, b1, W2, b2, tile_expert, tile_used):
    grid_spec = pltpu.PrefetchScalarGridSpec(
        num_scalar_prefetch=2,
        grid=(NTILES,),
        in_specs=[
            pl.BlockSpec((TILE, H), lambda j, te, us: (j, 0)),
            pl.BlockSpec((1, H, F), lambda j, te, us: (te[j], 0, 0)),
            pl.BlockSpec((1, 1, F), lambda j, te, us: (te[j], 0, 0)),
            pl.BlockSpec((1, F, H), lambda j, te, us: (te[j], 0, 0)),
            pl.BlockSpec((1, 1, H), lambda j, te, us: (te[j], 0, 0)),
        ],
        out_specs=pl.BlockSpec((TILE, H), lambda j, te, us: (j, 0)),
    )
    return pl.pallas_call(
        _ffn_body,
        grid_spec=grid_spec,
        out_shape=jax.ShapeDtypeStruct((NP, H), jnp.float32),
    )(tile_expert, tile_used, xs, W1, b1.reshape(E, 1, F), W2, b2.reshape(E, 1, H))


def kernel(x, ln1_w, ln1_b, Wq, bq, Wk, bk, Wv, bv, Wo, bo,
           ln2_w, ln2_b, Wr, br, W1, b1, W2, b2):
    x2d = x.reshape(S, H)
    q, k, v = _qkv(x2d, ln1_w, ln1_b, Wq, bq, Wk, bk, Wv, bv)
    o = _attn(q, k, v)
    x1, h2, logits = _post(x2d, o, Wo, bo, ln2_w, ln2_b, Wr, br)
    xs, pos, gates, tile_expert, tile_used = _route_sc(logits.T, h2)
    if _DEBUG_JNP_XS:
        xs = (jnp.zeros((NP, H), jnp.float32)
              .at[pos[0]].set(h2).at[pos[1]].set(h2))
    outp = _ffn(xs, W1, b1, W2, b2, tile_expert[:NTILES], tile_used[:NTILES])
    if _DEBUG_JNP_COMBINE:
        c1 = jnp.take(outp, pos[0], axis=0)
        c2 = jnp.take(outp, pos[1], axis=0)
        out = x1 + gates[0][:, None] * c1 + gates[1][:, None] * c2
    else:
        out = _combine_sc(outp, x1, gates, pos)
    return out.reshape(1, S, H)


# speedup vs baseline: 1.0367x; 1.0013x over previous
import functools

import jax
import jax.numpy as jnp
from jax import lax
from jax.experimental import pallas as pl
from jax.experimental.pallas import tpu as pltpu
from jax.experimental.pallas import tpu_sc as plsc

S, H, NH, DH = 2048, 768, 12, 64
E, F, TOPK = 8, 3072, 2
TILE = 256
NTILES = 24
NP = NTILES * TILE


def _qkv_body(x_ref, w_ref, b_ref, wq_ref, bq_ref, wk_ref, bk_ref,
              wv_ref, bv_ref, q_ref, k_ref, v_ref):
    x = x_ref[...]
    m = jnp.mean(x, axis=-1, keepdims=True)
    var = jnp.mean((x - m) ** 2, axis=-1, keepdims=True)
    h = (x - m) / jnp.sqrt(var + 1e-5) * w_ref[...] + b_ref[...]
    q_ref[...] = jnp.dot(h, wq_ref[...], preferred_element_type=jnp.float32) + bq_ref[...]
    k_ref[...] = jnp.dot(h, wk_ref[...], preferred_element_type=jnp.float32) + bk_ref[...]
    v_ref[...] = jnp.dot(h, wv_ref[...], preferred_element_type=jnp.float32) + bv_ref[...]


_RB = 512


def _qkv(x2d, ln1_w, ln1_b, Wq, bq, Wk, bk, Wv, bv):
    row = pl.BlockSpec((_RB, H), lambda j: (j, 0))
    vec = pl.BlockSpec((H,), lambda j: (0,))
    mat = pl.BlockSpec((H, H), lambda j: (0, 0))
    return pl.pallas_call(
        _qkv_body,
        grid=(S // _RB,),
        in_specs=[row, vec, vec, mat, vec, mat, vec, mat, vec],
        out_specs=[row, row, row],
        out_shape=[jax.ShapeDtypeStruct((S, H), jnp.float32)] * 3,
    )(x2d, ln1_w, ln1_b, Wq, bq, Wk, bk, Wv, bv)


_QB = 512


def _attn_body(q_ref, k_ref, v_ref, o_ref):
    for h in range(2):
        q = q_ref[:, h * DH:(h + 1) * DH]
        k = k_ref[:, h * DH:(h + 1) * DH]
        s = jax.lax.dot_general(q, k, (((1,), (1,)), ((), ())),
                                preferred_element_type=jnp.float32)
        s = s * (1.0 / 8.0)
        mx = jnp.max(s, axis=-1, keepdims=True)
        e = jnp.exp(s - mx)
        denom = jnp.sum(e, axis=-1, keepdims=True)
        o = jnp.dot(e, v_ref[:, h * DH:(h + 1) * DH],
                    preferred_element_type=jnp.float32)
        o_ref[:, h * DH:(h + 1) * DH] = o / denom


def _attn(q, k, v):
    kv_spec = pl.BlockSpec((S, 2 * DH), lambda j, qb: (0, j))
    q_spec = pl.BlockSpec((_QB, 2 * DH), lambda j, qb: (qb, j))
    return pl.pallas_call(
        _attn_body,
        grid=(NH // 2, S // _QB),
        in_specs=[q_spec, kv_spec, kv_spec],
        out_specs=q_spec,
        out_shape=jax.ShapeDtypeStruct((S, H), jnp.float32),
    )(q, k, v)


def _post_body(x_ref, o_ref, wo_ref, bo_ref, w_ref, b_ref, wr_ref, br_ref,
               x1_ref, h2_ref, lg_ref):
    x1 = x_ref[...] + jnp.dot(o_ref[...], wo_ref[...],
                              preferred_element_type=jnp.float32) + bo_ref[...]
    x1_ref[...] = x1
    m = jnp.mean(x1, axis=-1, keepdims=True)
    var = jnp.mean((x1 - m) ** 2, axis=-1, keepdims=True)
    h2 = (x1 - m) / jnp.sqrt(var + 1e-5) * w_ref[...] + b_ref[...]
    h2_ref[...] = h2
    lg_ref[...] = jnp.dot(h2, wr_ref[...], preferred_element_type=jnp.float32) + br_ref[...]


def _post(x2d, o, Wo, bo, ln2_w, ln2_b, Wr, br):
    row = pl.BlockSpec((_RB, H), lambda j: (j, 0))
    vec = pl.BlockSpec((H,), lambda j: (0,))
    return pl.pallas_call(
        _post_body,
        grid=(S // _RB,),
        in_specs=[row, row, pl.BlockSpec((H, H), lambda j: (0, 0)), vec,
                  vec, vec, pl.BlockSpec((H, E), lambda j: (0, 0)),
                  pl.BlockSpec((E,), lambda j: (0,))],
        out_specs=[row, row, pl.BlockSpec((_RB, E), lambda j: (j, 0))],
        out_shape=[
            jax.ShapeDtypeStruct((S, H), jnp.float32),
            jax.ShapeDtypeStruct((S, H), jnp.float32),
            jax.ShapeDtypeStruct((S, E), jnp.float32),
        ],
    )(x2d, o, Wo, bo, ln2_w, ln2_b, Wr, br)


TPW = S // 16


def _route_sc(logitsT, h2):
    mesh = plsc.VectorSubcoreMesh(core_axis_name="c", subcore_axis_name="s")

    @functools.partial(
        pl.kernel, mesh=mesh,
        out_type=[
            jax.ShapeDtypeStruct((NP, H), jnp.float32),
            jax.ShapeDtypeStruct((TOPK, S), jnp.int32),
            jax.ShapeDtypeStruct((TOPK, S), jnp.float32),
            jax.ShapeDtypeStruct((32,), jnp.int32),
            jax.ShapeDtypeStruct((32,), jnp.int32),
        ],
        compiler_params=pltpu.CompilerParams(
            needs_layout_passes=False, use_tc_tiling_on_sc=False),
        scratch_types=[
            pltpu.VMEM((E, TPW), jnp.float32),
            pltpu.VMEM((TOPK, TPW), jnp.int32),
            pltpu.VMEM((TOPK, TPW), jnp.int32),
            pltpu.VMEM((TOPK, TPW), jnp.float32),
            pltpu.VMEM((16,), jnp.int32),
            pltpu.VMEM((16, 16), jnp.int32),
            pltpu.VMEM((2, 32, H), jnp.float32),
            pltpu.VMEM((8, 32), jnp.int32),
            pltpu.VMEM((2, 16), jnp.int32),
            pltpu.VMEM((2, 16), jnp.int32),
            pltpu.VMEM_SHARED((16, 16), jnp.int32),
            pltpu.SemaphoreType.DMA,
        ])
    def route(logT_hbm, h2_hbm, xs_hbm, pos_hbm, g_hbm, te_hbm, us_hbm,
              lg_v, e_v, p_v, g_v, cnt_v, call_v, h2_v, idx_v, te_v, us_v,
              cnt_sh, sem):
        c = lax.axis_index("c")
        s = lax.axis_index("s")

        @pl.when(c == 0)
        def _body():
            t0 = s * TPW
            iota = lax.iota(jnp.int32, 16)
            for e in range(E):
                pltpu.sync_copy(logT_hbm.at[e, pl.ds(t0, TPW)], lg_v.at[e])
            cnt = [jnp.int32(0)] * E
            for g in range(TPW // 16):
                sl = pl.ds(g * 16, 16)
                vs = [lg_v[e, sl] for e in range(E)]
                m1 = vs[0]
                a1 = jnp.zeros((16,), jnp.int32)
                for e in range(1, E):
                    gt = vs[e] > m1
                    m1 = jnp.where(gt, vs[e], m1)
                    a1 = jnp.where(gt, e, a1)
                m2 = jnp.full((16,), -3e38, jnp.float32)
                a2 = jnp.zeros((16,), jnp.int32)
                for e in range(E):
                    ve = jnp.where(a1 == e, -3e38, vs[e])
                    gt = ve > m2
                    m2 = jnp.where(gt, ve, m2)
                    a2 = jnp.where(gt, e, a2)
                ed = jnp.exp(m2 - m1)
                den = 1.0 + ed
                e_v[0, sl] = a1
                e_v[1, sl] = a2
                g_v[0, sl] = 1.0 / den
                g_v[1, sl] = ed / den
                for e in range(E):
                    cnt[e] = cnt[e] + jnp.sum((a1 == e).astype(jnp.int32)) \
                                    + jnp.sum((a2 == e).astype(jnp.int32))
            cv = jnp.zeros((16,), jnp.int32)
            for e in range(E):
                cv = cv + jnp.where(iota == e, cnt[e], 0)
            cnt_v[...] = cv
            pltpu.sync_copy(cnt_v, cnt_sh.at[s])
            plsc.subcore_barrier()
            pltpu.sync_copy(cnt_sh, call_v)
            s_vec = jnp.zeros((16,), jnp.int32) + s
            total = jnp.zeros((16,), jnp.int32)
            before = jnp.zeros((16,), jnp.int32)
            for w2 in range(16):
                r = call_v[w2]
                total = total + r
                mask = (jnp.zeros((16,), jnp.int32) + w2) < s_vec
                before = before + jnp.where(mask, r, 0)
            padded = jnp.bitwise_and(total + (TILE - 1), ~(TILE - 1))
            incl = plsc.cumsum(padded)
            pstart = incl - padded
            base = pstart + before

            @pl.when(s == 0)
            def _tiles():
                tot = jnp.sum(jnp.where(iota == E - 1, incl, 0))
                for grp in range(2):
                    jv = (iota + grp * 16) * TILE
                    te = jnp.zeros((16,), jnp.int32)
                    for e in range(E):
                        ps_e = jnp.sum(jnp.where(iota == e, pstart, 0))
                        ie_e = jnp.sum(jnp.where(iota == e, incl, 0))
                        te = jnp.where((jv >= ps_e) & (jv < ie_e), e, te)
                    te_v[grp, :] = jnp.where(jv >= tot, E - 1, te)
                    us_v[grp, :] = (jv < tot).astype(jnp.int32)
                pltpu.sync_copy(te_v.at[0], te_hbm.at[pl.ds(0, 16)])
                pltpu.sync_copy(te_v.at[1], te_hbm.at[pl.ds(16, 16)])
                pltpu.sync_copy(us_v.at[0], us_hbm.at[pl.ds(0, 16)])
                pltpu.sync_copy(us_v.at[1], us_hbm.at[pl.ds(16, 16)])

            off = [jnp.int32(0)] * E
            for g in range(TPW // 16):
                sl = pl.ds(g * 16, 16)
                for j2 in range(TOPK):
                    ev = e_v[j2, sl]
                    p = jnp.zeros((16,), jnp.int32)
                    for e in range(E):
                        mask = ev == e
                        mi = mask.astype(jnp.int32)
                        ranks = plsc.cumsum(mi) - 1
                        base_e = jnp.sum(jnp.where(iota == e, base, 0))
                        p = jnp.where(mask, base_e + off[e] + ranks, p)
                        off[e] = off[e] + jnp.sum(mi)
                    p_v[j2, sl] = p
            pltpu.sync_copy(p_v.at[0], pos_hbm.at[0, pl.ds(t0, TPW)])
            pltpu.sync_copy(p_v.at[1], pos_hbm.at[1, pl.ds(t0, TPW)])
            pltpu.sync_copy(g_v.at[0], g_hbm.at[0, pl.ds(t0, TPW)])
            pltpu.sync_copy(g_v.at[1], g_hbm.at[1, pl.ds(t0, TPW)])
            handles = []
            for ch in range(TPW // 32):
                buf = h2_v.at[ch % 2]
                if ch >= 2:
                    handles.pop(0).wait()
                    handles.pop(0).wait()
                pltpu.sync_copy(h2_hbm.at[pl.ds(t0 + ch * 32, 32)], buf)
                for j2 in range(TOPK):
                    row = ch * 2 + j2
                    for hh in range(2):
                        idx_v[row, pl.ds(hh * 16, 16)] = \
                            p_v[j2, pl.ds(ch * 32 + hh * 16, 16)]
                    handles.append(
                        pltpu.async_copy(buf, xs_hbm.at[idx_v.at[row]], sem))
            for hc in handles:
                hc.wait()

    return route(logitsT, h2)


CPW = S // 32


def _combine_sc(outp, x1, gates, pos):
    mesh = plsc.VectorSubcoreMesh(core_axis_name="c", subcore_axis_name="s")

    @functools.partial(
        pl.kernel, mesh=mesh,
        out_type=jax.ShapeDtypeStruct((S, H), jnp.float32),
        compiler_params=pltpu.CompilerParams(needs_layout_passes=False),
        scratch_types=[
            pltpu.VMEM((TOPK, CPW), jnp.int32),
            pltpu.VMEM((TOPK, CPW), jnp.float32),
            pltpu.VMEM((2, 16), jnp.int32),
            pltpu.VMEM((2, 16), jnp.int32),
            pltpu.VMEM((2, 16, H), jnp.float32),
            pltpu.VMEM((2, 16, H), jnp.float32),
            pltpu.VMEM((16, H), jnp.float32),
            pltpu.VMEM((16, H), jnp.float32),
            pltpu.SemaphoreType.DMA,
        ])
    def combine(outp_hbm, x1_hbm, g_hbm, pos_hbm, out_hbm,
                pos_v, g_v, i1, i2, c1, c2, x1v, ov, sem):
        c = lax.axis_index("c")
        s = lax.axis_index("s")
        wid = s * 2 + c
        t0 = wid * CPW
        pltpu.sync_copy(pos_hbm.at[0, pl.ds(t0, CPW)], pos_v.at[0])
        pltpu.sync_copy(pos_hbm.at[1, pl.ds(t0, CPW)], pos_v.at[1])
        pltpu.sync_copy(g_hbm.at[0, pl.ds(t0, CPW)], g_v.at[0])
        pltpu.sync_copy(g_hbm.at[1, pl.ds(t0, CPW)], g_v.at[1])

        def fire(q):
            b = q % 2
            i1[b, :] = pos_v[0, pl.ds(q * 16, 16)]
            i2[b, :] = pos_v[1, pl.ds(q * 16, 16)]
            return [pltpu.async_copy(outp_hbm.at[i1.at[b]], c1.at[b], sem),
                    pltpu.async_copy(outp_hbm.at[i2.at[b]], c2.at[b], sem)]

        pend = fire(0)
        for q in range(CPW // 16):
            b = q % 2
            cur, pend = pend, (fire(q + 1) if q + 1 < CPW // 16 else [])
            for hc in cur:
                hc.wait()
            pltpu.sync_copy(x1_hbm.at[pl.ds(t0 + q * 16, 16)], x1v)
            gq1 = g_v[0, pl.ds(q * 16, 16)]
            gq2 = g_v[1, pl.ds(q * 16, 16)]
            for i in range(16):
                g1 = gq1[i]
                g2 = gq2[i]

                def body(kk, _):
                    rsl = pl.ds(kk * 16, 16)
                    ov[i, rsl] = (x1v[i, rsl] + g1 * c1[b, i, rsl]
                                  + g2 * c2[b, i, rsl])
                    return 0

                lax.fori_loop(0, H // 16, body, 0)
            pltpu.sync_copy(ov, out_hbm.at[pl.ds(t0 + q * 16, 16)])

    return combine(outp, x1, gates, pos)


def _ffn_body(te_ref, used_ref, xs_ref, w1_ref, b1_ref, w2_ref, b2_ref, out_ref):
    j = pl.program_id(0)

    @pl.when(used_ref[j] > 0)
    def _():
        xs = xs_ref[...]
        hid = jnp.dot(xs, w1_ref[0], preferred_element_type=jnp.float32) + b1_ref[0]
        act = jax.nn.gelu(hid)
        out_ref[...] = jnp.dot(act, w2_ref[0], preferred_element_type=jnp.float32) + b2_ref[0]


def _ffn(xs, W1, b1, W2, b2, tile_expert, tile_used):
    grid_spec = pltpu.PrefetchScalarGridSpec(
        num_scalar_prefetch=2,
        grid=(NTILES,),
        in_specs=[
            pl.BlockSpec((TILE, H), lambda j, te, us: (j, 0)),
            pl.BlockSpec((1, H, F), lambda j, te, us: (te[j], 0, 0)),
            pl.BlockSpec((1, 1, F), lambda j, te, us: (te[j], 0, 0)),
            pl.BlockSpec((1, F, H), lambda j, te, us: (te[j], 0, 0)),
            pl.BlockSpec((1, 1, H), lambda j, te, us: (te[j], 0, 0)),
        ],
        out_specs=pl.BlockSpec((TILE, H), lambda j, te, us: (j, 0)),
    )
    return pl.pallas_call(
        _ffn_body,
        grid_spec=grid_spec,
        out_shape=jax.ShapeDtypeStruct((NP, H), jnp.float32),
    )(tile_expert, tile_used, xs, W1, b1.reshape(E, 1, F), W2, b2.reshape(E, 1, H))


def kernel(x, ln1_w, ln1_b, Wq, bq, Wk, bk, Wv, bv, Wo, bo,
           ln2_w, ln2_b, Wr, br, W1, b1, W2, b2):
    x2d = x.reshape(S, H)
    q, k, v = _qkv(x2d, ln1_w, ln1_b, Wq, bq, Wk, bk, Wv, bv)
    o = _attn(q, k, v)
    x1, h2, logits = _post(x2d, o, Wo, bo, ln2_w, ln2_b, Wr, br)
    xs, pos, gates, tile_expert, tile_used = _route_sc(logits.T, h2)
    outp = _ffn(xs, W1, b1, W2, b2, tile_expert[:NTILES], tile_used[:NTILES])
    out = _combine_sc(outp, x1, gates, pos)
    return out.reshape(1, S, H)
